# baseline (speedup 1.0000x reference)
import functools
import numpy as np
import jax
import jax.numpy as jnp
from jax import lax
from jax.experimental import pallas as pl
from jax.experimental.pallas import tpu as pltpu
from jax.experimental.pallas import tpu_sc as plsc

N = 10000
E = 320000
D_IN = 128
H = 64
ED = 16
G = 256
NOUT = 1

NC = 2
NS = 16
NW = NC * NS
L = 16
K = 128
NCHUNK = E // K
CPW = -(-NCHUNK // NW)

f32 = jnp.float32
i32 = jnp.int32


def _mesh():
    return plsc.VectorSubcoreMesh(
        core_axis_name="c", subcore_axis_name="s", num_cores=NC, num_subcores=NS
    )


def _wid():
    return lax.axis_index("s") * NC + lax.axis_index("c")


def _leaky(v):
    return jnp.where(v >= 0, v, 0.01 * v)


CK = 80
NCH2 = E // CK
CW = NCH2 // NW
NSLOT = 5
DEPTH = 3
TOUT = CW // NSLOT


def _splat(v, jj):
    s = jnp.sum(jnp.where(lax.iota(i32, L) == jj, v, 0.0))
    return lax.broadcast_in_dim(s, (L,), ())


@functools.partial(
    pl.kernel,
    out_type=(
        jax.ShapeDtypeStruct((E // 2, 128), f32),
        jax.ShapeDtypeStruct((2, E // 2), f32),
    ),
    mesh=_mesh(),
    scratch_types=[
        pltpu.VMEM((CW, CK), i32),
        pltpu.VMEM((CW, CK), i32),
        pltpu.VMEM((N,), f32),
        pltpu.VMEM((CW * CK,), f32),
        pltpu.VMEM((NSLOT, CK, H), f32),
        pltpu.SemaphoreType.DMA,
    ] + [pltpu.SemaphoreType.DMA] * (2 * NSLOT),
    compiler_params=pltpu.CompilerParams(use_tc_tiling_on_sc=False, needs_layout_passes=False),
)
def _sc_row_gather(u_hbm, r_hbm, src2_hbm, dst2_hbm, uj_hbm, rd_hbm,
                   idxs, idxd, tbl, rdv, ring, semi, *sems):
    semg = sems[:NSLOT]
    semo = sems[NSLOT:]
    w = _wid()
    rows = pl.ds(w * CW, CW)
    waits = [
        pltpu.async_copy(src2_hbm.at[rows], idxs, semi),
        pltpu.async_copy(dst2_hbm.at[rows], idxd, semi),
        pltpu.async_copy(r_hbm, tbl, semi),
    ]
    for d in waits:
        d.wait()

    def g_desc(ci, b):
        return pltpu.make_async_copy(u_hbm.at[idxs.at[ci]], ring.at[b], semg[b])

    half = w // NS
    colbase = half * H

    def o_desc(ci, b):
        rowbase = (w * CW + ci) * CK - half * (E // 2)
        return pltpu.make_async_copy(
            ring.at[b],
            uj_hbm.at[pl.ds(rowbase, CK), pl.ds(colbase, H)],
            semo[b])

    for b in range(DEPTH):
        g_desc(b, b).start()

    @pl.loop(0, TOUT)
    def _outer(t):
        for b in range(NSLOT):
            i = t * NSLOT + b
            for j in range(CK // L):
                d16 = idxd[i, pl.ds(j * L, L)]
                rdv[pl.ds(i * CK + j * L, L)] = plsc.load_gather(tbl, [d16])
            g_desc(i, b).wait()
            o_desc(i, b).start()
            jn = i + DEPTH
            bn = (b + DEPTH) % NSLOT

            @pl.when(jn < CW)
            def _():
                @pl.when(i >= NSLOT - DEPTH)
                def _():
                    o_desc(jn - NSLOT, bn).wait()
                g_desc(jn, bn).start()

    for b in range(NSLOT):
        o_desc(CW - NSLOT + b, b).wait()
    pltpu.sync_copy(
        rdv, rd_hbm.at[half, pl.ds(w * CW * CK - half * (E // 2), CW * CK)])


def _make_scA(mode):
    assert mode in ("sum", "node")

    if mode == "sum":
        out_type = jax.ShapeDtypeStruct((NW, N), f32)
    else:
        out_type = (
            jax.ShapeDtypeStruct((E,), f32),
            jax.ShapeDtypeStruct((NW, N), f32),
        )
    scratch = [
        pltpu.VMEM((N,), f32),
        pltpu.VMEM((N,), f32),
        pltpu.VMEM((N,), f32),
        pltpu.VMEM((CW, CK), i32),
        pltpu.VMEM((CW, CK), i32),
        pltpu.VMEM((CW * CK,), f32),
        pltpu.SemaphoreType.DMA,
    ]

    def body(a_hbm, b_hbm, src2_hbm, dst2_hbm, *refs):
        if mode == "sum":
            (sp_hbm, tblA, tblB, acc, idxs, idxd, ebuf, sem) = refs
        else:
            (e_hbm, sp_hbm, tblA, tblB, acc, idxs, idxd, ebuf, sem) = refs
        w = _wid()
        rows = pl.ds(w * CW, CW)
        waits = [pltpu.async_copy(dst2_hbm.at[rows], idxd, sem)]
        if mode == "sum":
            waits.append(pltpu.async_copy(
                a_hbm.at[pl.ds(w * CW * CK, CW * CK)], ebuf, sem))
        else:
            waits.append(pltpu.async_copy(a_hbm, tblA, sem))
            waits.append(pltpu.async_copy(b_hbm, tblB, sem))
            waits.append(pltpu.async_copy(src2_hbm.at[rows], idxs, sem))

        @pl.loop(0, N // L)
        def _zero(j):
            acc[pl.ds(j * L, L)] = jnp.zeros((L,), f32)

        for d in waits:
            d.wait()

        @pl.loop(0, CW)
        def _chunks(ci):
            for j in range(CK // L):
                d16 = idxd[ci, pl.ds(j * L, L)]
                if mode == "sum":
                    ev = ebuf[pl.ds(ci * CK + j * L, L)]
                else:
                    s16 = idxs[ci, pl.ds(j * L, L)]
                    av = plsc.load_gather(tblA, [s16])
                    bv = plsc.load_gather(tblB, [d16])
                    ev = jnp.exp(_leaky(av + bv))
                    ebuf[pl.ds(ci * CK + j * L, L)] = ev
                plsc.addupdate_scatter(acc, [d16], ev)

        if mode == "node":
            pltpu.sync_copy(ebuf, e_hbm.at[pl.ds(w * CW * CK, CW * CK)])
        pltpu.sync_copy(acc, sp_hbm.at[w])

    return functools.partial(
        pl.kernel, out_type=out_type, mesh=_mesh(), scratch_types=scratch,
        compiler_params=pltpu.CompilerParams(use_tc_tiling_on_sc=False, needs_layout_passes=False),
    )(body)


_scA_sum = _make_scA("sum")
_scA_node = _make_scA("node")


_ZR = 125


def _make_scB(gather_rows):
    out_type = jax.ShapeDtypeStruct((NC, N, 128), f32)
    scratch = [
        pltpu.VMEM_SHARED((N, H), f32),
        pltpu.VMEM((CW, CK), i32),
        pltpu.VMEM((CW, CK), i32),
        pltpu.VMEM((CW * CK,), f32),
        pltpu.VMEM((NSLOT, CK, H), f32),
        pltpu.VMEM((_ZR, H), f32),
        pltpu.SemaphoreType.DMA,
    ] + [pltpu.SemaphoreType.DMA] * (2 * NSLOT)

    def body(rows_hbm, e_hbm, src2_hbm, dst2_hbm, hp_hbm,
             shacc, idxs, idxd, ebuf, ring, stage, semi, *sems):
        semg = sems[:NSLOT]
        sems_ = sems[NSLOT:]
        core = lax.axis_index("c")
        sid = lax.axis_index("s")
        w = sid * NC + core
        rows = pl.ds(w * CW, CW)

        waits = [
            pltpu.async_copy(dst2_hbm.at[rows], idxd, semi),
            pltpu.async_copy(e_hbm.at[pl.ds(w * CW * CK, CW * CK)], ebuf, semi),
        ]
        if gather_rows:
            waits.append(pltpu.async_copy(src2_hbm.at[rows], idxs, semi))

        @pl.loop(0, _ZR)
        def _z0(i):
            for q in range(H // L):
                stage[i, pl.ds(q * L, L)] = jnp.zeros((L,), f32)

        for k in range(N // NS // _ZR):
            pltpu.sync_copy(stage, shacc.at[pl.ds(sid * (N // NS) + k * _ZR, _ZR)])
        plsc.subcore_barrier()
        for d in waits:
            d.wait()

        half = sid // (NS // NC) if False else (w // NS)
        colbase = half * H

        def g_desc(ci, b):
            if gather_rows:
                return pltpu.make_async_copy(
                    rows_hbm.at[idxs.at[ci]], ring.at[b], semg[b])
            rowbase = (w * CW + ci) * CK - half * (E // 2)
            return pltpu.make_async_copy(
                rows_hbm.at[pl.ds(rowbase, CK), pl.ds(colbase, H)],
                ring.at[b], semg[b])

        def s_wait(ci, b):
            pltpu.make_async_copy(ring.at[b], shacc.at[idxd.at[ci]], sems_[b]).wait()

        for b in range(DEPTH):
            g_desc(b, b).start()

        @pl.loop(0, TOUT)
        def _outer(t):
            for b in range(NSLOT):
                i = t * NSLOT + b
                g_desc(i, b).wait()
                for j in range(CK // L):
                    coef = ebuf[pl.ds(i * CK + j * L, L)]
                    for jj in range(L):
                        spl = _splat(coef, jj)
                        r = j * L + jj
                        for q in range(H // L):
                            ring[b, r, pl.ds(q * L, L)] = (
                                ring[b, r, pl.ds(q * L, L)] * spl)
                pltpu.async_copy(ring.at[b], shacc.at[idxd.at[i]], sems_[b],
                                 add=True)
                jn = i + DEPTH
                bn = (b + DEPTH) % NSLOT

                @pl.when(jn < CW)
                def _():
                    @pl.when(i >= NSLOT - DEPTH)
                    def _():
                        s_wait(jn - NSLOT, bn)
                    g_desc(jn, bn).start()

        for b in range(NSLOT):
            s_wait(CW - NSLOT + b, b)

        plsc.subcore_barrier()
        for k in range(N // NS // _ZR):
            off = sid * (N // NS) + k * _ZR
            pltpu.sync_copy(shacc.at[pl.ds(off, _ZR)], stage)
            pltpu.sync_copy(stage, hp_hbm.at[core, pl.ds(off, _ZR), pl.ds(0, H)])

    return functools.partial(
        pl.kernel, out_type=out_type, mesh=_mesh(), scratch_types=scratch,
        compiler_params=pltpu.CompilerParams(use_tc_tiling_on_sc=False, needs_layout_passes=False),
    )(body)


_scB_linear = _make_scB(False)
_scB_gather = _make_scB(True)


def _gru_tc(h, x, WihT, WhhT, bih, bhh):
    gi = jnp.dot(h, WihT, preferred_element_type=f32) + bih
    gh = jnp.dot(x, WhhT, preferred_element_type=f32) + bhh
    ir, iz, ig = gi[:, :H], gi[:, H:2 * H], gi[:, 2 * H:]
    hr, hz, hg = gh[:, :H], gh[:, H:2 * H], gh[:, 2 * H:]
    r = jax.nn.sigmoid(ir + hr)
    z = jax.nn.sigmoid(iz + hz)
    n = jnp.tanh(ig + r * hg)
    return (1.0 - z) * n + z * x


def _elu(v):
    return jnp.where(v > 0, v, jnp.exp(jnp.minimum(v, 0.0)) - 1.0)


_BN1 = 2000


def _tc1_body(x_ref, wl_ref, bl_ref, wga_ref, attr_ref, x1_ref, u_ref, r_ref):
    x1 = _leaky(jnp.dot(x_ref[...], wl_ref[...], preferred_element_type=f32)
                + bl_ref[...])
    x1_ref[...] = x1
    u_ref[...] = jnp.dot(x1, wga_ref[...], preferred_element_type=f32)
    r_ref[...] = jnp.dot(x1, attr_ref[...], preferred_element_type=f32)


def _tc1(x, W_lin1, b_lin1, Wg1a, att_r):
    grid = (N // _BN1,)
    return pl.pallas_call(
        _tc1_body,
        grid=grid,
        in_specs=[
            pl.BlockSpec((_BN1, D_IN), lambda i: (i, 0)),
            pl.BlockSpec((D_IN, H), lambda i: (0, 0)),
            pl.BlockSpec((1, H), lambda i: (0, 0)),
            pl.BlockSpec((H, H), lambda i: (0, 0)),
            pl.BlockSpec((H, 1), lambda i: (0, 0)),
        ],
        out_specs=[
            pl.BlockSpec((_BN1, H), lambda i: (i, 0)),
            pl.BlockSpec((_BN1, H), lambda i: (i, 0)),
            pl.BlockSpec((_BN1, 1), lambda i: (i, 0)),
        ],
        out_shape=[
            jax.ShapeDtypeStruct((N, H), f32),
            jax.ShapeDtypeStruct((N, H), f32),
            jax.ShapeDtypeStruct((N, 1), f32),
        ],
    )(x, W_lin1, b_lin1.reshape(1, H), Wg1a, att_r.reshape(H, 1))


_BEP = 6400


def _tc2_body(ujp_ref, eap_ref, rd_ref, wgb2_ref, attl_ref,
              hjp_ref, ee_ref, eo_ref):
    i = pl.program_id(0)
    hjp = _leaky(ujp_ref[...]
                 + jnp.dot(eap_ref[...], wgb2_ref[...], preferred_element_type=f32))
    hjp_ref[...] = hjp
    q0 = jnp.sum(hjp[:, :H] * attl_ref[0][None, :], axis=1)
    q1 = jnp.sum(hjp[:, H:] * attl_ref[0][None, :], axis=1)
    sl = pl.ds(i * _BEP, _BEP)
    ee_ref[sl] = jnp.exp(_leaky(q0 + rd_ref[0, sl]))
    eo_ref[sl] = jnp.exp(_leaky(q1 + rd_ref[1, sl]))


def _tc2(ujp, eap, rd, Wg1b2, att_l):
    nb = E // 2 // _BEP
    return pl.pallas_call(
        _tc2_body,
        grid=(nb,),
        in_specs=[
            pl.BlockSpec((_BEP, 128), lambda i: (i, 0)),
            pl.BlockSpec((_BEP, 2 * ED), lambda i: (i, 0)),
            pl.BlockSpec((2, E // 2), lambda i: (0, 0)),
            pl.BlockSpec((2 * ED, 128), lambda i: (0, 0)),
            pl.BlockSpec((1, H), lambda i: (0, 0)),
        ],
        out_specs=[
            pl.BlockSpec((_BEP, 128), lambda i: (i, 0)),
            pl.BlockSpec((E // 2,), lambda i: (0,)),
            pl.BlockSpec((E // 2,), lambda i: (0,)),
        ],
        out_shape=[
            jax.ShapeDtypeStruct((E // 2, 128), f32),
            jax.ShapeDtypeStruct((E // 2,), f32),
            jax.ShapeDtypeStruct((E // 2,), f32),
        ],
    )(ujp, eap, rd, Wg1b2, att_l.reshape(1, H))


_BN3 = 2500


def _tc3_body(sp_ref, si_ref):
    s = jnp.sum(sp_ref[...], axis=0)
    si_ref[...] = (1.0 / (s + 1e-30))[:, None]


def _tc3(s_part):
    return pl.pallas_call(
        _tc3_body,
        out_shape=jax.ShapeDtypeStruct((N, 1), f32),
    )(s_part)


def _tc4_body(has_att, hp_ref, si_ref, xp_ref, w2_ref, b2_ref,
              wih_ref, whh_ref, bih_ref, bhh_ref, asrc_ref, adst_ref,
              xn_ref, ss_ref, sd_ref):
    hpre = (hp_ref[0, :, :H] + hp_ref[1, :, :H]) * si_ref[...]
    h = _elu(jnp.dot(hpre, w2_ref[...], preferred_element_type=f32) + b2_ref[...])
    xn = jnp.maximum(
        _gru_tc(h, xp_ref[...], wih_ref[...], whh_ref[...], bih_ref[...],
                bhh_ref[...]),
        0.0,
    )
    xn_ref[...] = xn
    if has_att:
        ss_ref[...] = jnp.dot(xn, asrc_ref[...], preferred_element_type=f32)
        sd_ref[...] = jnp.dot(xn, adst_ref[...], preferred_element_type=f32)
    else:
        ss_ref[...] = jnp.zeros_like(ss_ref)
        sd_ref[...] = jnp.zeros_like(sd_ref)


def _tc4(hp, sinv, x_prev, W2, b2, Wih, Whh, bih, bhh, a_src, a_dst, has_att):
    grid = (N // _BN1,)
    return pl.pallas_call(
        functools.partial(_tc4_body, has_att),
        grid=grid,
        in_specs=[
            pl.BlockSpec((NC, _BN1, 128), lambda i: (0, i, 0)),
            pl.BlockSpec((_BN1, 1), lambda i: (i, 0)),
            pl.BlockSpec((_BN1, H), lambda i: (i, 0)),
            pl.BlockSpec((H, H), lambda i: (0, 0)),
            pl.BlockSpec((1, H), lambda i: (0, 0)),
            pl.BlockSpec((H, 3 * H), lambda i: (0, 0)),
            pl.BlockSpec((H, 3 * H), lambda i: (0, 0)),
            pl.BlockSpec((1, 3 * H), lambda i: (0, 0)),
            pl.BlockSpec((1, 3 * H), lambda i: (0, 0)),
            pl.BlockSpec((H, 1), lambda i: (0, 0)),
            pl.BlockSpec((H, 1), lambda i: (0, 0)),
        ],
        out_specs=[
            pl.BlockSpec((_BN1, H), lambda i: (i, 0)),
            pl.BlockSpec((_BN1, 1), lambda i: (i, 0)),
            pl.BlockSpec((_BN1, 1), lambda i: (i, 0)),
        ],
        out_shape=[
            jax.ShapeDtypeStruct((N, H), f32),
            jax.ShapeDtypeStruct((N, 1), f32),
            jax.ShapeDtypeStruct((N, 1), f32),
        ],
    )(hp, sinv, x_prev, W2, b2.reshape(1, H), Wih.T, Whh.T,
      bih.reshape(1, 3 * H), bhh.reshape(1, 3 * H),
      a_src.reshape(H, 1), a_dst.reshape(H, 1))


_NBR = 2000


def _tcr_body(x_ref, b_ref, molw_ref, msrc_ref, mdst_ref, molb_ref,
              wih_ref, whh_ref, bih_ref, bhh_ref, wa_ref, ba_ref,
              wb_ref, bb_ref, out_ref):
    nblk = N // _NBR
    x = x_ref[...]
    bt = b_ref[...]
    iota_g = lax.broadcasted_iota(i32, (_NBR, G), 1)

    def onehot(nb):
        return (bt[nb * _NBR:(nb + 1) * _NBR] == iota_g).astype(f32)

    pool = jnp.zeros((G, H), f32)
    for nb in range(nblk):
        pool = pool + lax.dot_general(
            onehot(nb), x[nb * _NBR:(nb + 1) * _NBR],
            (((0,), (0,)), ((), ())), preferred_element_type=f32)
    out = jnp.maximum(pool, 0.0)

    hs = jnp.dot(x, molw_ref[...], preferred_element_type=f32)
    sn = jnp.dot(x, msrc_ref[...], preferred_element_type=f32)

    for _ in range(3):
        dg = jnp.dot(out, mdst_ref[...], preferred_element_type=f32)
        e_blks = []
        sb = jnp.zeros((G, 1), f32)
        for nb in range(nblk):
            oh = onehot(nb)
            z = _leaky(sn[nb * _NBR:(nb + 1) * _NBR]
                       + jnp.dot(oh, dg, preferred_element_type=f32))
            e = jnp.exp(z)
            e_blks.append(e)
            sb = sb + lax.dot_general(oh, e, (((0,), (0,)), ((), ())),
                                      preferred_element_type=f32)
        sbinv = 1.0 / (sb + 1e-30)
        hpool = jnp.zeros((G, H), f32)
        for nb in range(nblk):
            oh = onehot(nb)
            coef = e_blks[nb] * jnp.dot(oh, sbinv, preferred_element_type=f32)
            hpool = hpool + lax.dot_general(
                oh, hs[nb * _NBR:(nb + 1) * _NBR] * coef,
                (((0,), (0,)), ((), ())), preferred_element_type=f32)
        h = _elu(hpool + molb_ref[...])
        out = jnp.maximum(
            _gru_tc(h, out, wih_ref[...], whh_ref[...], bih_ref[...],
                    bhh_ref[...]),
            0.0,
        )

    o1 = jnp.maximum(jnp.dot(out, wa_ref[...], preferred_element_type=f32)
                     + ba_ref[...], 0.0)
    out_ref[...] = jnp.dot(o1, wb_ref[...], preferred_element_type=f32) + bb_ref[...]


def _tcr(x4, batch, mol_W, msrc, mdst, mol_b, Wih, Whh, bih, bhh,
         Wa, ba, Wb, bb):
    return pl.pallas_call(
        _tcr_body,
        out_shape=jax.ShapeDtypeStruct((G, NOUT), f32),
    )(x4, batch.reshape(N, 1), mol_W, msrc.reshape(H, 1), mdst.reshape(H, 1),
      mol_b.reshape(1, H), Wih.T, Whh.T, bih.reshape(1, 3 * H),
      bhh.reshape(1, 3 * H), Wa, ba.reshape(1, H), Wb, bb.reshape(1, NOUT))


def kernel(x, edge_index, edge_attr, batch,
           W_lin1, b_lin1, att_l, att_r, Wg1, Wg2, b_gate,
           gru0_Wih, gru0_Whh, gru0_bih, gru0_bhh,
           gru1_Wih, gru1_Whh, gru1_bih, gru1_bhh,
           gru2_Wih, gru2_Whh, gru2_bih, gru2_bhh,
           conv1_W, conv1_att_src, conv1_att_dst, conv1_b,
           conv2_W, conv2_att_src, conv2_att_dst, conv2_b,
           mol_W, mol_att_src, mol_att_dst, mol_b,
           molgru_Wih, molgru_Whh, molgru_bih, molgru_bhh,
           Wa, ba, Wb, bb):
    srcp = jnp.concatenate([edge_index[0, 0::2], edge_index[0, 1::2]])
    dstp = jnp.concatenate([edge_index[1, 0::2], edge_index[1, 1::2]])
    src2 = srcp.reshape(NCH2, CK)
    dst2 = dstp.reshape(NCH2, CK)

    x1, u, r = _tc1(x, W_lin1, b_lin1, Wg1[:H], att_r)
    uj, rd = _sc_row_gather(u, r.reshape(N), src2, dst2)
    Wg1b = Wg1[H:]
    zed = jnp.zeros((ED, H), f32)
    Wg1b2 = jnp.concatenate(
        [jnp.concatenate([Wg1b, zed], axis=1),
         jnp.concatenate([zed, Wg1b], axis=1)], axis=0)
    hjp, e_ev, e_od = _tc2(uj, edge_attr.reshape(E // 2, 2 * ED), rd,
                           Wg1b2, att_l)
    e0 = jnp.concatenate([e_ev, e_od])
    sp0 = _scA_sum(e0, e0, src2, dst2)
    sinv0 = _tc3(sp0)
    hp0 = _scB_linear(hjp, e0, src2, dst2)
    a_src1 = conv1_W @ conv1_att_src
    a_dst1 = conv1_W @ conv1_att_dst
    x2, ss1, sd1 = _tc4(hp0, sinv0, x1, Wg2, b_gate,
                        gru0_Wih, gru0_Whh, gru0_bih, gru0_bhh,
                        a_src1, a_dst1, True)

    e1, sp1 = _scA_node(ss1.reshape(N), sd1.reshape(N), src2, dst2)
    sinv1 = _tc3(sp1)
    hp1 = _scB_gather(x2, e1, src2, dst2)
    a_src2 = conv2_W @ conv2_att_src
    a_dst2 = conv2_W @ conv2_att_dst
    x3, ss2, sd2 = _tc4(hp1, sinv1, x2, conv1_W, conv1_b,
                        gru1_Wih, gru1_Whh, gru1_bih, gru1_bhh,
                        a_src2, a_dst2, True)

    e2, sp2 = _scA_node(ss2.reshape(N), sd2.reshape(N), src2, dst2)
    sinv2 = _tc3(sp2)
    hp2 = _scB_gather(x3, e2, src2, dst2)
    x4, _, _ = _tc4(hp2, sinv2, x3, conv2_W, conv2_b,
                    gru2_Wih, gru2_Whh, gru2_bih, gru2_bhh,
                    conv2_att_src, conv2_att_dst, False)

    msrc = mol_W @ mol_att_src
    mdst = mol_W @ mol_att_dst
    return _tcr(x4, batch, mol_W, msrc, mdst, mol_b,
                molgru_Wih, molgru_Whh, molgru_bih, molgru_bhh,
                Wa, ba, Wb, bb)


# speedup vs baseline: 1.2372x; 1.2372x over previous
import functools
import numpy as np
import jax
import jax.numpy as jnp
from jax import lax
from jax.experimental import pallas as pl
from jax.experimental.pallas import tpu as pltpu
from jax.experimental.pallas import tpu_sc as plsc

N = 10000
E = 320000
D_IN = 128
H = 64
ED = 16
G = 256
NOUT = 1

NC = 2
NS = 16
NW = NC * NS
L = 16
K = 128
NCHUNK = E // K
CPW = -(-NCHUNK // NW)

f32 = jnp.float32
i32 = jnp.int32


def _mesh():
    return plsc.VectorSubcoreMesh(
        core_axis_name="c", subcore_axis_name="s", num_cores=NC, num_subcores=NS
    )


def _wid():
    return lax.axis_index("s") * NC + lax.axis_index("c")


def _leaky(v):
    return jnp.where(v >= 0, v, 0.01 * v)


CK = 80
NCH2 = E // CK
CW = NCH2 // NW
NSLOT = 5
DEPTH = 3
TOUT = CW // NSLOT


def _splat(v, jj):
    s = jnp.sum(jnp.where(lax.iota(i32, L) == jj, v, 0.0))
    return lax.broadcast_in_dim(s, (L,), ())


@functools.partial(
    pl.kernel,
    out_type=(
        jax.ShapeDtypeStruct((E, 128), f32),
        jax.ShapeDtypeStruct((E,), f32),
    ),
    mesh=_mesh(),
    scratch_types=[
        pltpu.VMEM((CW, CK), i32),
        pltpu.VMEM((CW, CK), i32),
        pltpu.VMEM((N,), f32),
        pltpu.VMEM((CW * CK,), f32),
        pltpu.VMEM((NSLOT, CK, H), f32),
        pltpu.SemaphoreType.DMA,
    ] + [pltpu.SemaphoreType.DMA] * (2 * NSLOT),
    compiler_params=pltpu.CompilerParams(use_tc_tiling_on_sc=False, needs_layout_passes=False),
)
def _sc_row_gather(u_hbm, r_hbm, src2_hbm, dst2_hbm, uj_hbm, rd_hbm,
                   idxs, idxd, tbl, rdv, ring, semi, *sems):
    semg = sems[:NSLOT]
    semo = sems[NSLOT:]
    w = _wid()
    rows = pl.ds(w * CW, CW)
    waits = [
        pltpu.async_copy(src2_hbm.at[rows], idxs, semi),
        pltpu.async_copy(dst2_hbm.at[rows], idxd, semi),
        pltpu.async_copy(r_hbm, tbl, semi),
    ]
    for d in waits:
        d.wait()

    def g_desc(ci, b):
        return pltpu.make_async_copy(u_hbm.at[idxs.at[ci]], ring.at[b], semg[b])

    def o_desc(ci, b):
        return pltpu.make_async_copy(
            ring.at[b],
            uj_hbm.at[pl.ds((w * CW + ci) * CK, CK), pl.ds(0, H)],
            semo[b])

    for b in range(DEPTH):
        g_desc(b, b).start()

    @pl.loop(0, TOUT)
    def _outer(t):
        for b in range(NSLOT):
            i = t * NSLOT + b
            for j in range(CK // L):
                d16 = idxd[i, pl.ds(j * L, L)]
                rdv[pl.ds(i * CK + j * L, L)] = plsc.load_gather(tbl, [d16])
            g_desc(i, b).wait()
            o_desc(i, b).start()
            jn = i + DEPTH
            bn = (b + DEPTH) % NSLOT

            @pl.when(jn < CW)
            def _():
                @pl.when(i >= NSLOT - DEPTH)
                def _():
                    o_desc(jn - NSLOT, bn).wait()
                g_desc(jn, bn).start()

    for b in range(NSLOT):
        o_desc(CW - NSLOT + b, b).wait()
    pltpu.sync_copy(rdv, rd_hbm.at[pl.ds(w * CW * CK, CW * CK)])


def _make_scA(mode):
    assert mode in ("sum", "node")

    if mode == "sum":
        out_type = jax.ShapeDtypeStruct((NW, N), f32)
    else:
        out_type = (
            jax.ShapeDtypeStruct((E,), f32),
            jax.ShapeDtypeStruct((NW, N), f32),
        )
    scratch = [
        pltpu.VMEM((N,), f32),
        pltpu.VMEM((N,), f32),
        pltpu.VMEM((N,), f32),
        pltpu.VMEM((CW, CK), i32),
        pltpu.VMEM((CW, CK), i32),
        pltpu.VMEM((CW * CK,), f32),
        pltpu.SemaphoreType.DMA,
    ]

    def body(a_hbm, b_hbm, src2_hbm, dst2_hbm, *refs):
        if mode == "sum":
            (sp_hbm, tblA, tblB, acc, idxs, idxd, ebuf, sem) = refs
        else:
            (e_hbm, sp_hbm, tblA, tblB, acc, idxs, idxd, ebuf, sem) = refs
        w = _wid()
        rows = pl.ds(w * CW, CW)
        waits = [pltpu.async_copy(dst2_hbm.at[rows], idxd, sem)]
        if mode == "sum":
            waits.append(pltpu.async_copy(
                a_hbm.at[pl.ds(w * CW * CK, CW * CK)], ebuf, sem))
        else:
            waits.append(pltpu.async_copy(a_hbm, tblA, sem))
            waits.append(pltpu.async_copy(b_hbm, tblB, sem))
            waits.append(pltpu.async_copy(src2_hbm.at[rows], idxs, sem))

        @pl.loop(0, N // L)
        def _zero(j):
            acc[pl.ds(j * L, L)] = jnp.zeros((L,), f32)

        for d in waits:
            d.wait()

        @pl.loop(0, CW)
        def _chunks(ci):
            for j in range(CK // L):
                d16 = idxd[ci, pl.ds(j * L, L)]
                if mode == "sum":
                    ev = ebuf[pl.ds(ci * CK + j * L, L)]
                else:
                    s16 = idxs[ci, pl.ds(j * L, L)]
                    av = plsc.load_gather(tblA, [s16])
                    bv = plsc.load_gather(tblB, [d16])
                    ev = jnp.exp(_leaky(av + bv))
                    ebuf[pl.ds(ci * CK + j * L, L)] = ev
                plsc.addupdate_scatter(acc, [d16], ev)

        if mode == "node":
            pltpu.sync_copy(ebuf, e_hbm.at[pl.ds(w * CW * CK, CW * CK)])
        pltpu.sync_copy(acc, sp_hbm.at[w])

    return functools.partial(
        pl.kernel, out_type=out_type, mesh=_mesh(), scratch_types=scratch,
        compiler_params=pltpu.CompilerParams(use_tc_tiling_on_sc=False, needs_layout_passes=False),
    )(body)


_scA_sum = _make_scA("sum")
_scA_node = _make_scA("node")


_ZR = 125


def _make_scB(gather_rows):
    out_type = jax.ShapeDtypeStruct((NC, N, 128), f32)
    scratch = [
        pltpu.VMEM_SHARED((N, H), f32),
        pltpu.VMEM((CW, CK), i32),
        pltpu.VMEM((CW, CK), i32),
        pltpu.VMEM((CW * CK,), f32),
        pltpu.VMEM((NSLOT, CK, H), f32),
        pltpu.VMEM((_ZR, H), f32),
        pltpu.SemaphoreType.DMA,
    ] + [pltpu.SemaphoreType.DMA] * (2 * NSLOT)

    def body(rows_hbm, e_hbm, src2_hbm, dst2_hbm, hp_hbm,
             shacc, idxs, idxd, ebuf, ring, stage, semi, *sems):
        semg = sems[:NSLOT]
        sems_ = sems[NSLOT:]
        core = lax.axis_index("c")
        sid = lax.axis_index("s")
        w = sid * NC + core
        rows = pl.ds(w * CW, CW)

        waits = [
            pltpu.async_copy(dst2_hbm.at[rows], idxd, semi),
            pltpu.async_copy(e_hbm.at[pl.ds(w * CW * CK, CW * CK)], ebuf, semi),
        ]
        if gather_rows:
            waits.append(pltpu.async_copy(src2_hbm.at[rows], idxs, semi))

        @pl.loop(0, _ZR)
        def _z0(i):
            for q in range(H // L):
                stage[i, pl.ds(q * L, L)] = jnp.zeros((L,), f32)

        for k in range(N // NS // _ZR):
            pltpu.sync_copy(stage, shacc.at[pl.ds(sid * (N // NS) + k * _ZR, _ZR)])
        plsc.subcore_barrier()
        for d in waits:
            d.wait()

        def g_desc(ci, b):
            if gather_rows:
                return pltpu.make_async_copy(
                    rows_hbm.at[idxs.at[ci]], ring.at[b], semg[b])
            return pltpu.make_async_copy(
                rows_hbm.at[pl.ds((w * CW + ci) * CK, CK), pl.ds(0, H)],
                ring.at[b], semg[b])

        def s_wait(ci, b):
            pltpu.make_async_copy(ring.at[b], shacc.at[idxd.at[ci]], sems_[b]).wait()

        for b in range(DEPTH):
            g_desc(b, b).start()

        @pl.loop(0, TOUT)
        def _outer(t):
            for b in range(NSLOT):
                i = t * NSLOT + b
                g_desc(i, b).wait()
                for j in range(CK // L):
                    coef = ebuf[pl.ds(i * CK + j * L, L)]
                    for jj in range(L):
                        spl = _splat(coef, jj)
                        r = j * L + jj
                        for q in range(H // L):
                            ring[b, r, pl.ds(q * L, L)] = (
                                ring[b, r, pl.ds(q * L, L)] * spl)
                pltpu.async_copy(ring.at[b], shacc.at[idxd.at[i]], sems_[b],
                                 add=True)
                jn = i + DEPTH
                bn = (b + DEPTH) % NSLOT

                @pl.when(jn < CW)
                def _():
                    @pl.when(i >= NSLOT - DEPTH)
                    def _():
                        s_wait(jn - NSLOT, bn)
                    g_desc(jn, bn).start()

        for b in range(NSLOT):
            s_wait(CW - NSLOT + b, b)

        plsc.subcore_barrier()
        for k in range(N // NS // _ZR):
            off = sid * (N // NS) + k * _ZR
            pltpu.sync_copy(shacc.at[pl.ds(off, _ZR)], stage)
            pltpu.sync_copy(stage, hp_hbm.at[core, pl.ds(off, _ZR), pl.ds(0, H)])

    return functools.partial(
        pl.kernel, out_type=out_type, mesh=_mesh(), scratch_types=scratch,
        compiler_params=pltpu.CompilerParams(use_tc_tiling_on_sc=False, needs_layout_passes=False),
    )(body)


_scB_linear = _make_scB(False)
_scB_gather = _make_scB(True)


def _gru_tc(h, x, WihT, WhhT, bih, bhh):
    gi = jnp.dot(h, WihT, preferred_element_type=f32) + bih
    gh = jnp.dot(x, WhhT, preferred_element_type=f32) + bhh
    ir, iz, ig = gi[:, :H], gi[:, H:2 * H], gi[:, 2 * H:]
    hr, hz, hg = gh[:, :H], gh[:, H:2 * H], gh[:, 2 * H:]
    r = jax.nn.sigmoid(ir + hr)
    z = jax.nn.sigmoid(iz + hz)
    n = jnp.tanh(ig + r * hg)
    return (1.0 - z) * n + z * x


def _elu(v):
    return jnp.where(v > 0, v, jnp.exp(jnp.minimum(v, 0.0)) - 1.0)


_BN1 = 2000


def _tc1_body(x_ref, wl_ref, bl_ref, wga_ref, attr_ref, x1_ref, u_ref, r_ref):
    x1 = _leaky(jnp.dot(x_ref[...], wl_ref[...], preferred_element_type=f32)
                + bl_ref[...])
    x1_ref[...] = x1
    u_ref[...] = jnp.dot(x1, wga_ref[...], preferred_element_type=f32)
    r_ref[...] = jnp.dot(x1, attr_ref[...], preferred_element_type=f32)


def _tc1(x, W_lin1, b_lin1, Wg1a, att_r):
    grid = (N // _BN1,)
    return pl.pallas_call(
        _tc1_body,
        grid=grid,
        in_specs=[
            pl.BlockSpec((_BN1, D_IN), lambda i: (i, 0)),
            pl.BlockSpec((D_IN, H), lambda i: (0, 0)),
            pl.BlockSpec((1, H), lambda i: (0, 0)),
            pl.BlockSpec((H, H), lambda i: (0, 0)),
            pl.BlockSpec((H, 1), lambda i: (0, 0)),
        ],
        out_specs=[
            pl.BlockSpec((_BN1, H), lambda i: (i, 0)),
            pl.BlockSpec((_BN1, H), lambda i: (i, 0)),
            pl.BlockSpec((_BN1, 1), lambda i: (i, 0)),
        ],
        out_shape=[
            jax.ShapeDtypeStruct((N, H), f32),
            jax.ShapeDtypeStruct((N, H), f32),
            jax.ShapeDtypeStruct((N, 1), f32),
        ],
    )(x, W_lin1, b_lin1.reshape(1, H), Wg1a, att_r.reshape(H, 1))


_BE2 = 8192


def _tc2_body(ujp_ref, ea_ref, rd_ref, wgb_ref, attl_ref, hjp_ref, e_ref):
    hj = _leaky(ujp_ref[:, :H]
                + jnp.dot(ea_ref[...], wgb_ref[...], preferred_element_type=f32))
    hjp_ref[:, :H] = hj
    z = _leaky(jnp.sum(hj * attl_ref[0][None, :], axis=1) + rd_ref[...])
    e_ref[...] = jnp.exp(z)


def _tc2(ujp, ea, rd, Wg1b, att_l):
    grid = (-(-E // _BE2),)
    return pl.pallas_call(
        _tc2_body,
        grid=grid,
        in_specs=[
            pl.BlockSpec((_BE2, 128), lambda i: (i, 0)),
            pl.BlockSpec((_BE2, ED), lambda i: (i, 0)),
            pl.BlockSpec((_BE2,), lambda i: (i,)),
            pl.BlockSpec((ED, H), lambda i: (0, 0)),
            pl.BlockSpec((1, H), lambda i: (0, 0)),
        ],
        out_specs=[
            pl.BlockSpec((_BE2, 128), lambda i: (i, 0)),
            pl.BlockSpec((_BE2,), lambda i: (i,)),
        ],
        out_shape=[
            jax.ShapeDtypeStruct((E, 128), f32),
            jax.ShapeDtypeStruct((E,), f32),
        ],
    )(ujp, ea, rd, Wg1b, att_l.reshape(1, H))


_BN3 = 2500


def _tc3_body(sp_ref, si_ref):
    s = jnp.sum(sp_ref[...], axis=0)
    si_ref[...] = (1.0 / (s + 1e-30))[:, None]


def _tc3(s_part):
    return pl.pallas_call(
        _tc3_body,
        out_shape=jax.ShapeDtypeStruct((N, 1), f32),
    )(s_part)


def _tc4_body(has_att, hp_ref, si_ref, xp_ref, w2_ref, b2_ref,
              wih_ref, whh_ref, bih_ref, bhh_ref, asrc_ref, adst_ref,
              xn_ref, ss_ref, sd_ref):
    hpre = (hp_ref[0, :, :H] + hp_ref[1, :, :H]) * si_ref[...]
    h = _elu(jnp.dot(hpre, w2_ref[...], preferred_element_type=f32) + b2_ref[...])
    xn = jnp.maximum(
        _gru_tc(h, xp_ref[...], wih_ref[...], whh_ref[...], bih_ref[...],
                bhh_ref[...]),
        0.0,
    )
    xn_ref[...] = xn
    if has_att:
        ss_ref[...] = jnp.dot(xn, asrc_ref[...], preferred_element_type=f32)
        sd_ref[...] = jnp.dot(xn, adst_ref[...], preferred_element_type=f32)
    else:
        ss_ref[...] = jnp.zeros_like(ss_ref)
        sd_ref[...] = jnp.zeros_like(sd_ref)


def _tc4(hp, sinv, x_prev, W2, b2, Wih, Whh, bih, bhh, a_src, a_dst, has_att):
    grid = (N // _BN1,)
    return pl.pallas_call(
        functools.partial(_tc4_body, has_att),
        grid=grid,
        in_specs=[
            pl.BlockSpec((NC, _BN1, 128), lambda i: (0, i, 0)),
            pl.BlockSpec((_BN1, 1), lambda i: (i, 0)),
            pl.BlockSpec((_BN1, H), lambda i: (i, 0)),
            pl.BlockSpec((H, H), lambda i: (0, 0)),
            pl.BlockSpec((1, H), lambda i: (0, 0)),
            pl.BlockSpec((H, 3 * H), lambda i: (0, 0)),
            pl.BlockSpec((H, 3 * H), lambda i: (0, 0)),
            pl.BlockSpec((1, 3 * H), lambda i: (0, 0)),
            pl.BlockSpec((1, 3 * H), lambda i: (0, 0)),
            pl.BlockSpec((H, 1), lambda i: (0, 0)),
            pl.BlockSpec((H, 1), lambda i: (0, 0)),
        ],
        out_specs=[
            pl.BlockSpec((_BN1, H), lambda i: (i, 0)),
            pl.BlockSpec((_BN1, 1), lambda i: (i, 0)),
            pl.BlockSpec((_BN1, 1), lambda i: (i, 0)),
        ],
        out_shape=[
            jax.ShapeDtypeStruct((N, H), f32),
            jax.ShapeDtypeStruct((N, 1), f32),
            jax.ShapeDtypeStruct((N, 1), f32),
        ],
    )(hp, sinv, x_prev, W2, b2.reshape(1, H), Wih.T, Whh.T,
      bih.reshape(1, 3 * H), bhh.reshape(1, 3 * H),
      a_src.reshape(H, 1), a_dst.reshape(H, 1))


_NBR = 2000


def _tcr_body(x_ref, b_ref, molw_ref, msrc_ref, mdst_ref, molb_ref,
              wih_ref, whh_ref, bih_ref, bhh_ref, wa_ref, ba_ref,
              wb_ref, bb_ref, out_ref):
    nblk = N // _NBR
    x = x_ref[...]
    bt = b_ref[...]
    iota_g = lax.broadcasted_iota(i32, (_NBR, G), 1)

    def onehot(nb):
        return (bt[nb * _NBR:(nb + 1) * _NBR] == iota_g).astype(f32)

    pool = jnp.zeros((G, H), f32)
    for nb in range(nblk):
        pool = pool + lax.dot_general(
            onehot(nb), x[nb * _NBR:(nb + 1) * _NBR],
            (((0,), (0,)), ((), ())), preferred_element_type=f32)
    out = jnp.maximum(pool, 0.0)

    hs = jnp.dot(x, molw_ref[...], preferred_element_type=f32)
    sn = jnp.dot(x, msrc_ref[...], preferred_element_type=f32)

    for _ in range(3):
        dg = jnp.dot(out, mdst_ref[...], preferred_element_type=f32)
        e_blks = []
        sb = jnp.zeros((G, 1), f32)
        for nb in range(nblk):
            oh = onehot(nb)
            z = _leaky(sn[nb * _NBR:(nb + 1) * _NBR]
                       + jnp.dot(oh, dg, preferred_element_type=f32))
            e = jnp.exp(z)
            e_blks.append(e)
            sb = sb + lax.dot_general(oh, e, (((0,), (0,)), ((), ())),
                                      preferred_element_type=f32)
        sbinv = 1.0 / (sb + 1e-30)
        hpool = jnp.zeros((G, H), f32)
        for nb in range(nblk):
            oh = onehot(nb)
            coef = e_blks[nb] * jnp.dot(oh, sbinv, preferred_element_type=f32)
            hpool = hpool + lax.dot_general(
                oh, hs[nb * _NBR:(nb + 1) * _NBR] * coef,
                (((0,), (0,)), ((), ())), preferred_element_type=f32)
        h = _elu(hpool + molb_ref[...])
        out = jnp.maximum(
            _gru_tc(h, out, wih_ref[...], whh_ref[...], bih_ref[...],
                    bhh_ref[...]),
            0.0,
        )

    o1 = jnp.maximum(jnp.dot(out, wa_ref[...], preferred_element_type=f32)
                     + ba_ref[...], 0.0)
    out_ref[...] = jnp.dot(o1, wb_ref[...], preferred_element_type=f32) + bb_ref[...]


def _tcr(x4, batch, mol_W, msrc, mdst, mol_b, Wih, Whh, bih, bhh,
         Wa, ba, Wb, bb):
    return pl.pallas_call(
        _tcr_body,
        out_shape=jax.ShapeDtypeStruct((G, NOUT), f32),
    )(x4, batch.reshape(N, 1), mol_W, msrc.reshape(H, 1), mdst.reshape(H, 1),
      mol_b.reshape(1, H), Wih.T, Whh.T, bih.reshape(1, 3 * H),
      bhh.reshape(1, 3 * H), Wa, ba.reshape(1, H), Wb, bb.reshape(1, NOUT))


def kernel(x, edge_index, edge_attr, batch,
           W_lin1, b_lin1, att_l, att_r, Wg1, Wg2, b_gate,
           gru0_Wih, gru0_Whh, gru0_bih, gru0_bhh,
           gru1_Wih, gru1_Whh, gru1_bih, gru1_bhh,
           gru2_Wih, gru2_Whh, gru2_bih, gru2_bhh,
           conv1_W, conv1_att_src, conv1_att_dst, conv1_b,
           conv2_W, conv2_att_src, conv2_att_dst, conv2_b,
           mol_W, mol_att_src, mol_att_dst, mol_b,
           molgru_Wih, molgru_Whh, molgru_bih, molgru_bhh,
           Wa, ba, Wb, bb):
    src2 = edge_index[0].reshape(NCH2, CK)
    dst2 = edge_index[1].reshape(NCH2, CK)

    x1, u, r = _tc1(x, W_lin1, b_lin1, Wg1[:H], att_r)
    uj, rd = _sc_row_gather(u, r.reshape(N), src2, dst2)
    hjp, e0 = _tc2(uj, edge_attr, rd, Wg1[H:], att_l)
    sp0 = _scA_sum(e0, e0, src2, dst2)
    sinv0 = _tc3(sp0)
    hp0 = _scB_linear(hjp, e0, src2, dst2)
    a_src1 = conv1_W @ conv1_att_src
    a_dst1 = conv1_W @ conv1_att_dst
    x2, ss1, sd1 = _tc4(hp0, sinv0, x1, Wg2, b_gate,
                        gru0_Wih, gru0_Whh, gru0_bih, gru0_bhh,
                        a_src1, a_dst1, True)

    e1, sp1 = _scA_node(ss1.reshape(N), sd1.reshape(N), src2, dst2)
    sinv1 = _tc3(sp1)
    hp1 = _scB_gather(x2, e1, src2, dst2)
    a_src2 = conv2_W @ conv2_att_src
    a_dst2 = conv2_W @ conv2_att_dst
    x3, ss2, sd2 = _tc4(hp1, sinv1, x2, conv1_W, conv1_b,
                        gru1_Wih, gru1_Whh, gru1_bih, gru1_bhh,
                        a_src2, a_dst2, True)

    e2, sp2 = _scA_node(ss2.reshape(N), sd2.reshape(N), src2, dst2)
    sinv2 = _tc3(sp2)
    hp2 = _scB_gather(x3, e2, src2, dst2)
    x4, _, _ = _tc4(hp2, sinv2, x3, conv2_W, conv2_b,
                    gru2_Wih, gru2_Whh, gru2_bih, gru2_bhh,
                    conv2_att_src, conv2_att_dst, False)

    msrc = mol_W @ mol_att_src
    mdst = mol_W @ mol_att_dst
    return _tcr(x4, batch, mol_W, msrc, mdst, mol_b,
                molgru_Wih, molgru_Whh, molgru_bih, molgru_bhh,
                Wa, ba, Wb, bb)


# speedup vs baseline: 1.2627x; 1.0206x over previous
import functools
import numpy as np
import jax
import jax.numpy as jnp
from jax import lax
from jax.experimental import pallas as pl
from jax.experimental.pallas import tpu as pltpu
from jax.experimental.pallas import tpu_sc as plsc

N = 10000
E = 320000
D_IN = 128
H = 64
ED = 16
G = 256
NOUT = 1

NC = 2
NS = 16
NW = NC * NS
L = 16
K = 128
NCHUNK = E // K
CPW = -(-NCHUNK // NW)

f32 = jnp.float32
i32 = jnp.int32


def _mesh():
    return plsc.VectorSubcoreMesh(
        core_axis_name="c", subcore_axis_name="s", num_cores=NC, num_subcores=NS
    )


def _wid():
    return lax.axis_index("s") * NC + lax.axis_index("c")


def _leaky(v):
    return jnp.where(v >= 0, v, 0.01 * v)


CK = 80
NCH2 = E // CK
CW = NCH2 // NW
NSLOT = 5
DEPTH = 3
TOUT = CW // NSLOT


def _splat(v, jj):
    idx = lax.iota(i32, L) * 0 + jj
    dn = lax.GatherDimensionNumbers(
        offset_dims=(), collapsed_slice_dims=(0,), start_index_map=(0,),
        operand_batching_dims=(), start_indices_batching_dims=())
    return lax.gather(v, idx[:, None], dn, (1,),
                      mode=lax.GatherScatterMode.PROMISE_IN_BOUNDS)


@functools.partial(
    pl.kernel,
    out_type=(
        jax.ShapeDtypeStruct((E, 128), f32),
        jax.ShapeDtypeStruct((E,), f32),
    ),
    mesh=_mesh(),
    scratch_types=[
        pltpu.VMEM((CW, CK), i32),
        pltpu.VMEM((CW, CK), i32),
        pltpu.VMEM((N,), f32),
        pltpu.VMEM((CW * CK,), f32),
        pltpu.VMEM((NSLOT, CK, H), f32),
        pltpu.SemaphoreType.DMA,
    ] + [pltpu.SemaphoreType.DMA] * (2 * NSLOT),
    compiler_params=pltpu.CompilerParams(use_tc_tiling_on_sc=False, needs_layout_passes=False),
)
def _sc_row_gather(u_hbm, r_hbm, src2_hbm, dst2_hbm, uj_hbm, rd_hbm,
                   idxs, idxd, tbl, rdv, ring, semi, *sems):
    semg = sems[:NSLOT]
    semo = sems[NSLOT:]
    w = _wid()
    rows = pl.ds(w * CW, CW)
    waits = [
        pltpu.async_copy(src2_hbm.at[rows], idxs, semi),
        pltpu.async_copy(dst2_hbm.at[rows], idxd, semi),
        pltpu.async_copy(r_hbm, tbl, semi),
    ]
    for d in waits:
        d.wait()

    def g_desc(ci, b):
        return pltpu.make_async_copy(u_hbm.at[idxs.at[ci]], ring.at[b], semg[b])

    def o_desc(ci, b):
        return pltpu.make_async_copy(
            ring.at[b],
            uj_hbm.at[pl.ds((w * CW + ci) * CK, CK), pl.ds(0, H)],
            semo[b])

    for b in range(DEPTH):
        g_desc(b, b).start()

    @pl.loop(0, TOUT)
    def _outer(t):
        for b in range(NSLOT):
            i = t * NSLOT + b
            for j in range(CK // L):
                d16 = idxd[i, pl.ds(j * L, L)]
                rdv[pl.ds(i * CK + j * L, L)] = plsc.load_gather(tbl, [d16])
            g_desc(i, b).wait()
            o_desc(i, b).start()
            jn = i + DEPTH
            bn = (b + DEPTH) % NSLOT

            @pl.when(jn < CW)
            def _():
                @pl.when(i >= NSLOT - DEPTH)
                def _():
                    o_desc(jn - NSLOT, bn).wait()
                g_desc(jn, bn).start()

    for b in range(NSLOT):
        o_desc(CW - NSLOT + b, b).wait()
    pltpu.sync_copy(rdv, rd_hbm.at[pl.ds(w * CW * CK, CW * CK)])


def _make_scA(mode):
    assert mode in ("sum", "node")

    if mode == "sum":
        out_type = jax.ShapeDtypeStruct((NW, N), f32)
    else:
        out_type = (
            jax.ShapeDtypeStruct((E,), f32),
            jax.ShapeDtypeStruct((NW, N), f32),
        )
    scratch = [
        pltpu.VMEM((N,), f32),
        pltpu.VMEM((N,), f32),
        pltpu.VMEM((N,), f32),
        pltpu.VMEM((CW, CK), i32),
        pltpu.VMEM((CW, CK), i32),
        pltpu.VMEM((CW * CK,), f32),
        pltpu.SemaphoreType.DMA,
    ]

    def body(a_hbm, b_hbm, src2_hbm, dst2_hbm, *refs):
        if mode == "sum":
            (sp_hbm, tblA, tblB, acc, idxs, idxd, ebuf, sem) = refs
        else:
            (e_hbm, sp_hbm, tblA, tblB, acc, idxs, idxd, ebuf, sem) = refs
        w = _wid()
        rows = pl.ds(w * CW, CW)
        waits = [pltpu.async_copy(dst2_hbm.at[rows], idxd, sem)]
        if mode == "sum":
            waits.append(pltpu.async_copy(
                a_hbm.at[pl.ds(w * CW * CK, CW * CK)], ebuf, sem))
        else:
            waits.append(pltpu.async_copy(a_hbm, tblA, sem))
            waits.append(pltpu.async_copy(b_hbm, tblB, sem))
            waits.append(pltpu.async_copy(src2_hbm.at[rows], idxs, sem))

        @pl.loop(0, N // L)
        def _zero(j):
            acc[pl.ds(j * L, L)] = jnp.zeros((L,), f32)

        for d in waits:
            d.wait()

        @pl.loop(0, CW)
        def _chunks(ci):
            for j in range(CK // L):
                d16 = idxd[ci, pl.ds(j * L, L)]
                if mode == "sum":
                    ev = ebuf[pl.ds(ci * CK + j * L, L)]
                else:
                    s16 = idxs[ci, pl.ds(j * L, L)]
                    av = plsc.load_gather(tblA, [s16])
                    bv = plsc.load_gather(tblB, [d16])
                    ev = jnp.exp(_leaky(av + bv))
                    ebuf[pl.ds(ci * CK + j * L, L)] = ev
                plsc.addupdate_scatter(acc, [d16], ev)

        if mode == "node":
            pltpu.sync_copy(ebuf, e_hbm.at[pl.ds(w * CW * CK, CW * CK)])
        pltpu.sync_copy(acc, sp_hbm.at[w])

    return functools.partial(
        pl.kernel, out_type=out_type, mesh=_mesh(), scratch_types=scratch,
        compiler_params=pltpu.CompilerParams(use_tc_tiling_on_sc=False, needs_layout_passes=False),
    )(body)


_scA_sum = _make_scA("sum")
_scA_node = _make_scA("node")


_ZR = 125


def _make_scB(gather_rows):
    out_type = jax.ShapeDtypeStruct((NC, N, 128), f32)
    scratch = [
        pltpu.VMEM_SHARED((N, H), f32),
        pltpu.VMEM((CW, CK), i32),
        pltpu.VMEM((CW, CK), i32),
        pltpu.VMEM((CW * CK,), f32),
        pltpu.VMEM((NSLOT, CK, H), f32),
        pltpu.VMEM((_ZR, H), f32),
        pltpu.SemaphoreType.DMA,
    ] + [pltpu.SemaphoreType.DMA] * (2 * NSLOT)

    def body(rows_hbm, e_hbm, src2_hbm, dst2_hbm, hp_hbm,
             shacc, idxs, idxd, ebuf, ring, stage, semi, *sems):
        semg = sems[:NSLOT]
        sems_ = sems[NSLOT:]
        core = lax.axis_index("c")
        sid = lax.axis_index("s")
        w = sid * NC + core
        rows = pl.ds(w * CW, CW)

        waits = [
            pltpu.async_copy(dst2_hbm.at[rows], idxd, semi),
            pltpu.async_copy(e_hbm.at[pl.ds(w * CW * CK, CW * CK)], ebuf, semi),
        ]
        if gather_rows:
            waits.append(pltpu.async_copy(src2_hbm.at[rows], idxs, semi))

        @pl.loop(0, _ZR)
        def _z0(i):
            for q in range(H // L):
                stage[i, pl.ds(q * L, L)] = jnp.zeros((L,), f32)

        for k in range(N // NS // _ZR):
            pltpu.sync_copy(stage, shacc.at[pl.ds(sid * (N // NS) + k * _ZR, _ZR)])
        plsc.subcore_barrier()
        for d in waits:
            d.wait()

        def g_desc(ci, b):
            if gather_rows:
                return pltpu.make_async_copy(
                    rows_hbm.at[idxs.at[ci]], ring.at[b], semg[b])
            return pltpu.make_async_copy(
                rows_hbm.at[pl.ds((w * CW + ci) * CK, CK), pl.ds(0, H)],
                ring.at[b], semg[b])

        def s_wait(ci, b):
            pltpu.make_async_copy(ring.at[b], shacc.at[idxd.at[ci]], sems_[b]).wait()

        for b in range(DEPTH):
            g_desc(b, b).start()

        @pl.loop(0, TOUT)
        def _outer(t):
            for b in range(NSLOT):
                i = t * NSLOT + b
                g_desc(i, b).wait()
                for j in range(CK // L):
                    coef = ebuf[pl.ds(i * CK + j * L, L)]
                    for jj in range(L):
                        spl = _splat(coef, jj)
                        r = j * L + jj
                        for q in range(H // L):
                            ring[b, r, pl.ds(q * L, L)] = (
                                ring[b, r, pl.ds(q * L, L)] * spl)
                pltpu.async_copy(ring.at[b], shacc.at[idxd.at[i]], sems_[b],
                                 add=True)
                jn = i + DEPTH
                bn = (b + DEPTH) % NSLOT

                @pl.when(jn < CW)
                def _():
                    @pl.when(i >= NSLOT - DEPTH)
                    def _():
                        s_wait(jn - NSLOT, bn)
                    g_desc(jn, bn).start()

        for b in range(NSLOT):
            s_wait(CW - NSLOT + b, b)

        plsc.subcore_barrier()
        for k in range(N // NS // _ZR):
            off = sid * (N // NS) + k * _ZR
            pltpu.sync_copy(shacc.at[pl.ds(off, _ZR)], stage)
            pltpu.sync_copy(stage, hp_hbm.at[core, pl.ds(off, _ZR), pl.ds(0, H)])

    return functools.partial(
        pl.kernel, out_type=out_type, mesh=_mesh(), scratch_types=scratch,
        compiler_params=pltpu.CompilerParams(use_tc_tiling_on_sc=False, needs_layout_passes=False),
    )(body)


_scB_linear = _make_scB(False)
_scB_gather = _make_scB(True)


def _gru_tc(h, x, WihT, WhhT, bih, bhh):
    gi = jnp.dot(h, WihT, preferred_element_type=f32) + bih
    gh = jnp.dot(x, WhhT, preferred_element_type=f32) + bhh
    ir, iz, ig = gi[:, :H], gi[:, H:2 * H], gi[:, 2 * H:]
    hr, hz, hg = gh[:, :H], gh[:, H:2 * H], gh[:, 2 * H:]
    r = jax.nn.sigmoid(ir + hr)
    z = jax.nn.sigmoid(iz + hz)
    n = jnp.tanh(ig + r * hg)
    return (1.0 - z) * n + z * x


def _elu(v):
    return jnp.where(v > 0, v, jnp.exp(jnp.minimum(v, 0.0)) - 1.0)


_BN1 = 2000


def _tc1_body(x_ref, wl_ref, bl_ref, wga_ref, attr_ref, x1_ref, u_ref, r_ref):
    x1 = _leaky(jnp.dot(x_ref[...], wl_ref[...], preferred_element_type=f32)
                + bl_ref[...])
    x1_ref[...] = x1
    u_ref[...] = jnp.dot(x1, wga_ref[...], preferred_element_type=f32)
    r_ref[...] = jnp.dot(x1, attr_ref[...], preferred_element_type=f32)


def _tc1(x, W_lin1, b_lin1, Wg1a, att_r):
    grid = (N // _BN1,)
    return pl.pallas_call(
        _tc1_body,
        grid=grid,
        in_specs=[
            pl.BlockSpec((_BN1, D_IN), lambda i: (i, 0)),
            pl.BlockSpec((D_IN, H), lambda i: (0, 0)),
            pl.BlockSpec((1, H), lambda i: (0, 0)),
            pl.BlockSpec((H, H), lambda i: (0, 0)),
            pl.BlockSpec((H, 1), lambda i: (0, 0)),
        ],
        out_specs=[
            pl.BlockSpec((_BN1, H), lambda i: (i, 0)),
            pl.BlockSpec((_BN1, H), lambda i: (i, 0)),
            pl.BlockSpec((_BN1, 1), lambda i: (i, 0)),
        ],
        out_shape=[
            jax.ShapeDtypeStruct((N, H), f32),
            jax.ShapeDtypeStruct((N, H), f32),
            jax.ShapeDtypeStruct((N, 1), f32),
        ],
    )(x, W_lin1, b_lin1.reshape(1, H), Wg1a, att_r.reshape(H, 1))


_BE2 = 8192


def _tc2_body(ujp_ref, ea_ref, rd_ref, wgb_ref, attl_ref, hjp_ref, e_ref):
    hj = _leaky(ujp_ref[:, :H]
                + jnp.dot(ea_ref[...], wgb_ref[...], preferred_element_type=f32))
    hjp_ref[:, :H] = hj
    z = _leaky(jnp.sum(hj * attl_ref[0][None, :], axis=1) + rd_ref[...])
    e_ref[...] = jnp.exp(z)


def _tc2(ujp, ea, rd, Wg1b, att_l):
    grid = (-(-E // _BE2),)
    return pl.pallas_call(
        _tc2_body,
        grid=grid,
        in_specs=[
            pl.BlockSpec((_BE2, 128), lambda i: (i, 0)),
            pl.BlockSpec((_BE2, ED), lambda i: (i, 0)),
            pl.BlockSpec((_BE2,), lambda i: (i,)),
            pl.BlockSpec((ED, H), lambda i: (0, 0)),
            pl.BlockSpec((1, H), lambda i: (0, 0)),
        ],
        out_specs=[
            pl.BlockSpec((_BE2, 128), lambda i: (i, 0)),
            pl.BlockSpec((_BE2,), lambda i: (i,)),
        ],
        out_shape=[
            jax.ShapeDtypeStruct((E, 128), f32),
            jax.ShapeDtypeStruct((E,), f32),
        ],
    )(ujp, ea, rd, Wg1b, att_l.reshape(1, H))


_BN3 = 2500


def _tc3_body(sp_ref, si_ref):
    s = jnp.sum(sp_ref[...], axis=0)
    si_ref[...] = (1.0 / (s + 1e-30))[:, None]


def _tc3(s_part):
    return pl.pallas_call(
        _tc3_body,
        out_shape=jax.ShapeDtypeStruct((N, 1), f32),
    )(s_part)


def _tc4_body(has_att, hp_ref, si_ref, xp_ref, w2_ref, b2_ref,
              wih_ref, whh_ref, bih_ref, bhh_ref, asrc_ref, adst_ref,
              xn_ref, ss_ref, sd_ref):
    hpre = (hp_ref[0, :, :H] + hp_ref[1, :, :H]) * si_ref[...]
    h = _elu(jnp.dot(hpre, w2_ref[...], preferred_element_type=f32) + b2_ref[...])
    xn = jnp.maximum(
        _gru_tc(h, xp_ref[...], wih_ref[...], whh_ref[...], bih_ref[...],
                bhh_ref[...]),
        0.0,
    )
    xn_ref[...] = xn
    if has_att:
        ss_ref[...] = jnp.dot(xn, asrc_ref[...], preferred_element_type=f32)
        sd_ref[...] = jnp.dot(xn, adst_ref[...], preferred_element_type=f32)
    else:
        ss_ref[...] = jnp.zeros_like(ss_ref)
        sd_ref[...] = jnp.zeros_like(sd_ref)


def _tc4(hp, sinv, x_prev, W2, b2, Wih, Whh, bih, bhh, a_src, a_dst, has_att):
    grid = (N // _BN1,)
    return pl.pallas_call(
        functools.partial(_tc4_body, has_att),
        grid=grid,
        in_specs=[
            pl.BlockSpec((NC, _BN1, 128), lambda i: (0, i, 0)),
            pl.BlockSpec((_BN1, 1), lambda i: (i, 0)),
            pl.BlockSpec((_BN1, H), lambda i: (i, 0)),
            pl.BlockSpec((H, H), lambda i: (0, 0)),
            pl.BlockSpec((1, H), lambda i: (0, 0)),
            pl.BlockSpec((H, 3 * H), lambda i: (0, 0)),
            pl.BlockSpec((H, 3 * H), lambda i: (0, 0)),
            pl.BlockSpec((1, 3 * H), lambda i: (0, 0)),
            pl.BlockSpec((1, 3 * H), lambda i: (0, 0)),
            pl.BlockSpec((H, 1), lambda i: (0, 0)),
            pl.BlockSpec((H, 1), lambda i: (0, 0)),
        ],
        out_specs=[
            pl.BlockSpec((_BN1, H), lambda i: (i, 0)),
            pl.BlockSpec((_BN1, 1), lambda i: (i, 0)),
            pl.BlockSpec((_BN1, 1), lambda i: (i, 0)),
        ],
        out_shape=[
            jax.ShapeDtypeStruct((N, H), f32),
            jax.ShapeDtypeStruct((N, 1), f32),
            jax.ShapeDtypeStruct((N, 1), f32),
        ],
    )(hp, sinv, x_prev, W2, b2.reshape(1, H), Wih.T, Whh.T,
      bih.reshape(1, 3 * H), bhh.reshape(1, 3 * H),
      a_src.reshape(H, 1), a_dst.reshape(H, 1))


_NBR = 2000


def _tcr_body(x_ref, b_ref, molw_ref, msrc_ref, mdst_ref, molb_ref,
              wih_ref, whh_ref, bih_ref, bhh_ref, wa_ref, ba_ref,
              wb_ref, bb_ref, out_ref):
    nblk = N // _NBR
    x = x_ref[...]
    bt = b_ref[...]
    iota_g = lax.broadcasted_iota(i32, (_NBR, G), 1)

    def onehot(nb):
        return (bt[nb * _NBR:(nb + 1) * _NBR] == iota_g).astype(f32)

    pool = jnp.zeros((G, H), f32)
    for nb in range(nblk):
        pool = pool + lax.dot_general(
            onehot(nb), x[nb * _NBR:(nb + 1) * _NBR],
            (((0,), (0,)), ((), ())), preferred_element_type=f32)
    out = jnp.maximum(pool, 0.0)

    hs = jnp.dot(x, molw_ref[...], preferred_element_type=f32)
    sn = jnp.dot(x, msrc_ref[...], preferred_element_type=f32)

    for _ in range(3):
        dg = jnp.dot(out, mdst_ref[...], preferred_element_type=f32)
        e_blks = []
        sb = jnp.zeros((G, 1), f32)
        for nb in range(nblk):
            oh = onehot(nb)
            z = _leaky(sn[nb * _NBR:(nb + 1) * _NBR]
                       + jnp.dot(oh, dg, preferred_element_type=f32))
            e = jnp.exp(z)
            e_blks.append(e)
            sb = sb + lax.dot_general(oh, e, (((0,), (0,)), ((), ())),
                                      preferred_element_type=f32)
        sbinv = 1.0 / (sb + 1e-30)
        hpool = jnp.zeros((G, H), f32)
        for nb in range(nblk):
            oh = onehot(nb)
            coef = e_blks[nb] * jnp.dot(oh, sbinv, preferred_element_type=f32)
            hpool = hpool + lax.dot_general(
                oh, hs[nb * _NBR:(nb + 1) * _NBR] * coef,
                (((0,), (0,)), ((), ())), preferred_element_type=f32)
        h = _elu(hpool + molb_ref[...])
        out = jnp.maximum(
            _gru_tc(h, out, wih_ref[...], whh_ref[...], bih_ref[...],
                    bhh_ref[...]),
            0.0,
        )

    o1 = jnp.maximum(jnp.dot(out, wa_ref[...], preferred_element_type=f32)
                     + ba_ref[...], 0.0)
    out_ref[...] = jnp.dot(o1, wb_ref[...], preferred_element_type=f32) + bb_ref[...]


def _tcr(x4, batch, mol_W, msrc, mdst, mol_b, Wih, Whh, bih, bhh,
         Wa, ba, Wb, bb):
    return pl.pallas_call(
        _tcr_body,
        out_shape=jax.ShapeDtypeStruct((G, NOUT), f32),
    )(x4, batch.reshape(N, 1), mol_W, msrc.reshape(H, 1), mdst.reshape(H, 1),
      mol_b.reshape(1, H), Wih.T, Whh.T, bih.reshape(1, 3 * H),
      bhh.reshape(1, 3 * H), Wa, ba.reshape(1, H), Wb, bb.reshape(1, NOUT))


def kernel(x, edge_index, edge_attr, batch,
           W_lin1, b_lin1, att_l, att_r, Wg1, Wg2, b_gate,
           gru0_Wih, gru0_Whh, gru0_bih, gru0_bhh,
           gru1_Wih, gru1_Whh, gru1_bih, gru1_bhh,
           gru2_Wih, gru2_Whh, gru2_bih, gru2_bhh,
           conv1_W, conv1_att_src, conv1_att_dst, conv1_b,
           conv2_W, conv2_att_src, conv2_att_dst, conv2_b,
           mol_W, mol_att_src, mol_att_dst, mol_b,
           molgru_Wih, molgru_Whh, molgru_bih, molgru_bhh,
           Wa, ba, Wb, bb):
    src2 = edge_index[0].reshape(NCH2, CK)
    dst2 = edge_index[1].reshape(NCH2, CK)

    x1, u, r = _tc1(x, W_lin1, b_lin1, Wg1[:H], att_r)
    uj, rd = _sc_row_gather(u, r.reshape(N), src2, dst2)
    hjp, e0 = _tc2(uj, edge_attr, rd, Wg1[H:], att_l)
    sp0 = _scA_sum(e0, e0, src2, dst2)
    sinv0 = _tc3(sp0)
    hp0 = _scB_linear(hjp, e0, src2, dst2)
    a_src1 = conv1_W @ conv1_att_src
    a_dst1 = conv1_W @ conv1_att_dst
    x2, ss1, sd1 = _tc4(hp0, sinv0, x1, Wg2, b_gate,
                        gru0_Wih, gru0_Whh, gru0_bih, gru0_bhh,
                        a_src1, a_dst1, True)

    e1, sp1 = _scA_node(ss1.reshape(N), sd1.reshape(N), src2, dst2)
    sinv1 = _tc3(sp1)
    hp1 = _scB_gather(x2, e1, src2, dst2)
    a_src2 = conv2_W @ conv2_att_src
    a_dst2 = conv2_W @ conv2_att_dst
    x3, ss2, sd2 = _tc4(hp1, sinv1, x2, conv1_W, conv1_b,
                        gru1_Wih, gru1_Whh, gru1_bih, gru1_bhh,
                        a_src2, a_dst2, True)

    e2, sp2 = _scA_node(ss2.reshape(N), sd2.reshape(N), src2, dst2)
    sinv2 = _tc3(sp2)
    hp2 = _scB_gather(x3, e2, src2, dst2)
    x4, _, _ = _tc4(hp2, sinv2, x3, conv2_W, conv2_b,
                    gru2_Wih, gru2_Whh, gru2_bih, gru2_bhh,
                    conv2_att_src, conv2_att_dst, False)

    msrc = mol_W @ mol_att_src
    mdst = mol_W @ mol_att_dst
    return _tcr(x4, batch, mol_W, msrc, mdst, mol_b,
                molgru_Wih, molgru_Whh, molgru_bih, molgru_bhh,
                Wa, ba, Wb, bb)
